# baseline (speedup 1.0000x reference)
import functools

import jax
import jax.numpy as jnp
from jax import lax
from jax.experimental import pallas as pl
from jax.experimental.pallas import tpu as pltpu

T = 4
F = 256
N = 10000
E = 160000
NP = 10240
EB = 1280
NB = 512


def _msg_body(t0_ref, t1_ref, wa_ref, wb_ref, bm_ref, out_ref):
    acc = jnp.dot(wa_ref[...], t0_ref[...], preferred_element_type=jnp.float32, precision=jax.lax.Precision.HIGHEST)
    acc += jnp.dot(wb_ref[...], t1_ref[...], preferred_element_type=jnp.float32, precision=jax.lax.Precision.HIGHEST)
    out_ref[...] = acc + bm_ref[...]


def _msg_matmul(tmp0_t, tmp1_t, WmA, WmB, bm2):
    grid = (E // EB,)
    return pl.pallas_call(
        _msg_body,
        grid=grid,
        in_specs=[
            pl.BlockSpec((F, EB), lambda j: (0, j)),
            pl.BlockSpec((F, EB), lambda j: (0, j)),
            pl.BlockSpec((F, F), lambda j: (0, 0)),
            pl.BlockSpec((F, F), lambda j: (0, 0)),
            pl.BlockSpec((F, 1), lambda j: (0, 0)),
        ],
        out_specs=pl.BlockSpec((F, EB), lambda j: (0, j)),
        out_shape=jax.ShapeDtypeStruct((F, E), jnp.float32),
    )(tmp0_t, tmp1_t, WmA, WmB, bm2)


def _gru_body(agg_ref, st_ref, wih_ref, whh_ref, bih_ref, bhh_ref, out_ref):
    gi = jnp.dot(wih_ref[...], agg_ref[...], preferred_element_type=jnp.float32, precision=jax.lax.Precision.HIGHEST)
    gi += bih_ref[...]
    gh = jnp.dot(whh_ref[...], st_ref[...], preferred_element_type=jnp.float32, precision=jax.lax.Precision.HIGHEST)
    gh += bhh_ref[...]
    r = jax.nn.sigmoid(gi[0:F, :] + gh[0:F, :])
    z = jax.nn.sigmoid(gi[F:2 * F, :] + gh[F:2 * F, :])
    n = jnp.tanh(gi[2 * F:, :] + r * gh[2 * F:, :])
    out_ref[...] = (1.0 - z) * n + z * st_ref[...]


def _gru(agg_t, state_t, W_ih, W_hh, bih2, bhh2):
    grid = (NP // NB,)
    return pl.pallas_call(
        _gru_body,
        grid=grid,
        in_specs=[
            pl.BlockSpec((F, NB), lambda j: (0, j)),
            pl.BlockSpec((F, NB), lambda j: (0, j)),
            pl.BlockSpec((3 * F, F), lambda j: (0, 0)),
            pl.BlockSpec((3 * F, F), lambda j: (0, 0)),
            pl.BlockSpec((3 * F, 1), lambda j: (0, 0)),
            pl.BlockSpec((3 * F, 1), lambda j: (0, 0)),
        ],
        out_specs=pl.BlockSpec((F, NB), lambda j: (0, j)),
        out_shape=jax.ShapeDtypeStruct((F, NP), jnp.float32),
    )(agg_t, state_t, W_ih, W_hh, bih2, bhh2)


def _head_body(st_ref, w1_ref, b1_ref, w2_ref, b2_ref, w3_ref, b3_ref,
               out_ref, acc_ref):
    j = pl.program_id(0)
    nb = pl.num_programs(0)

    col = jax.lax.broadcasted_iota(jnp.int32, (NB, 1), 0) + j * NB
    ones = jnp.where(col < N, 1.0, 0.0).astype(jnp.float32)
    part = jnp.dot(st_ref[...], ones, preferred_element_type=jnp.float32, precision=jax.lax.Precision.HIGHEST)

    @pl.when(j == 0)
    def _():
        acc_ref[...] = jnp.zeros_like(acc_ref)

    acc_ref[...] += part

    @pl.when(j == nb - 1)
    def _():
        feat = acc_ref[...]
        h1 = jnp.maximum(jnp.dot(w1_ref[...], feat,
                                 preferred_element_type=jnp.float32, precision=jax.lax.Precision.HIGHEST)
                         + b1_ref[...], 0.0)
        h2 = jnp.maximum(jnp.dot(w2_ref[...], h1,
                                 preferred_element_type=jnp.float32, precision=jax.lax.Precision.HIGHEST)
                         + b2_ref[...], 0.0)
        out_ref[...] = (jnp.dot(w3_ref[...], h2,
                                preferred_element_type=jnp.float32, precision=jax.lax.Precision.HIGHEST)
                        + b3_ref[...])


def _head(state_t, W1, b12, W2, b22, W3, b32):
    grid = (NP // NB,)
    out = pl.pallas_call(
        _head_body,
        grid=grid,
        in_specs=[
            pl.BlockSpec((F, NB), lambda j: (0, j)),
            pl.BlockSpec((F, F), lambda j: (0, 0)),
            pl.BlockSpec((F, 1), lambda j: (0, 0)),
            pl.BlockSpec((F // 2, F), lambda j: (0, 0)),
            pl.BlockSpec((F // 2, 1), lambda j: (0, 0)),
            pl.BlockSpec((1, F // 2), lambda j: (0, 0)),
            pl.BlockSpec((1, 1), lambda j: (0, 0)),
        ],
        out_specs=pl.BlockSpec((1, 1), lambda j: (0, 0)),
        out_shape=jax.ShapeDtypeStruct((1, 1), jnp.float32),
        scratch_shapes=[pltpu.VMEM((F, 1), jnp.float32)],
    )(state_t, W1, b12, W2, b22, W3, b32)
    return out.reshape((1,))


def kernel(link_state, pair, Wm, bm, W_ih, W_hh, b_ih, b_hh, W1, b1, W2, b2, W3, b3):
    idx0 = pair[0::2].T.astype(jnp.int32)
    idx1 = pair[1::2].T.astype(jnp.int32)
    WmA = Wm[:, :F]
    WmB = Wm[:, F:]
    bm2 = bm.reshape(F, 1)
    bih2 = b_ih.reshape(3 * F, 1)
    bhh2 = b_hh.reshape(3 * F, 1)
    b12 = b1.reshape(F, 1)
    b22 = b2.reshape(F // 2, 1)
    b32 = b3.reshape(1, 1)

    state_t = jnp.pad(link_state.T, ((0, 0), (0, NP - N)))
    rows = jnp.arange(F, dtype=jnp.int32)[:, None]

    for _ in range(T):
        tmp0_t = jnp.take_along_axis(state_t[:, :N], idx0, axis=1)
        tmp1_t = jnp.take_along_axis(state_t[:, :N], idx1, axis=1)
        m_t = _msg_matmul(tmp0_t, tmp1_t, WmA, WmB, bm2)
        agg_t = jnp.zeros((F, NP), jnp.float32).at[rows, idx0].add(m_t)
        state_t = _gru(agg_t, state_t, W_ih, W_hh, bih2, bhh2)

    return _head(state_t, W1, b12, W2, b22, W3, b32)


# speedup vs baseline: 538.3776x; 538.3776x over previous
import functools

import jax
import jax.numpy as jnp
from jax import lax
from jax.experimental import pallas as pl
from jax.experimental.pallas import tpu as pltpu
from jax.experimental.pallas import tpu_sc as plsc

T = 4
F = 256
N = 10000
E = 160000

NW = 32
RPW = 8
C = 640
CB = RPW * C
NCH = E // C
NP = 10240
RN = RPW * NP
NB = 1280
EBLK = C


def _sc_mesh():
    return plsc.VectorSubcoreMesh(core_axis_name="c", subcore_axis_name="s",
                                  num_cores=2, num_subcores=16)


_SC_PARAMS = pltpu.CompilerParams(needs_layout_passes=False)


def _sc_gather(state_flat, i0_blk, i1_blk):
    def body(st_hbm, i0_hbm, i1_hbm, t0_hbm, t1_hbm,
             st_v, i0a, i0b, i1a, i1b, o0a, o0b, o1a, o1b,
             s_st, s_i0, s_i1, s_o0, s_o1):
        wid = lax.axis_index("s") * 2 + lax.axis_index("c")
        base = wid * NCH * CB
        ib0 = (i0a, i0b)
        ib1 = (i1a, i1b)
        ob0 = (o0a, o0b)
        ob1 = (o1a, o1b)

        def idx_start(c, b):
            off = base + c * CB
            pltpu.async_copy(i0_hbm.at[pl.ds(off, CB)], ib0[b], s_i0.at[b])
            pltpu.async_copy(i1_hbm.at[pl.ds(off, CB)], ib1[b], s_i1.at[b])

        def idx_wait(b):
            pltpu.make_async_copy(i0_hbm.at[pl.ds(0, CB)], ib0[b], s_i0.at[b]).wait()
            pltpu.make_async_copy(i1_hbm.at[pl.ds(0, CB)], ib1[b], s_i1.at[b]).wait()

        def out_start(c, b):
            off = base + c * CB
            pltpu.async_copy(ob0[b], t0_hbm.at[pl.ds(off, CB)], s_o0.at[b])
            pltpu.async_copy(ob1[b], t1_hbm.at[pl.ds(off, CB)], s_o1.at[b])

        def out_wait(b):
            pltpu.make_async_copy(ob0[b], t0_hbm.at[pl.ds(0, CB)], s_o0.at[b]).wait()
            pltpu.make_async_copy(ob1[b], t1_hbm.at[pl.ds(0, CB)], s_o1.at[b]).wait()

        pltpu.async_copy(st_hbm.at[pl.ds(wid * RN, RN)], st_v, s_st)
        idx_start(0, 0)
        pltpu.make_async_copy(st_hbm.at[pl.ds(0, RN)], st_v, s_st).wait()

        def compute(b):
            for r in range(RPW):
                rbase = jnp.int32(r * NP)

                def g_body(g, _, r=r, rbase=rbase, b=b):
                    off = r * C + g * 16
                    iv0 = ib0[b][pl.ds(off, 16)] + rbase
                    ob0[b][pl.ds(off, 16)] = plsc.load_gather(st_v, [iv0])
                    iv1 = ib1[b][pl.ds(off, 16)] + rbase
                    ob1[b][pl.ds(off, 16)] = plsc.load_gather(st_v, [iv1])
                    return 0

                lax.fori_loop(0, C // 16, g_body, 0)

        def loop_body(c2, _):
            for b in (0, 1):
                c = c2 * 2 + b
                idx_wait(b)

                @pl.when(c < NCH - 1)
                def _(c=c, b=b):
                    idx_start(c + 1, 1 - b)

                @pl.when(c2 >= 1)
                def _(b=b):
                    out_wait(b)

                compute(b)
                out_start(c, b)
            return 0

        lax.fori_loop(0, NCH // 2, loop_body, 0)
        out_wait(0)
        out_wait(1)

    f32 = jnp.float32
    i32 = jnp.int32
    run = pl.kernel(
        body,
        out_type=[jax.ShapeDtypeStruct((F * E,), f32),
                  jax.ShapeDtypeStruct((F * E,), f32)],
        mesh=_sc_mesh(),
        compiler_params=_SC_PARAMS,
        scratch_types=[
            pltpu.VMEM((RN,), f32),
            pltpu.VMEM((CB,), i32), pltpu.VMEM((CB,), i32),
            pltpu.VMEM((CB,), i32), pltpu.VMEM((CB,), i32),
            pltpu.VMEM((CB,), f32), pltpu.VMEM((CB,), f32),
            pltpu.VMEM((CB,), f32), pltpu.VMEM((CB,), f32),
            pltpu.SemaphoreType.DMA,
            pltpu.SemaphoreType.DMA((2,)), pltpu.SemaphoreType.DMA((2,)),
            pltpu.SemaphoreType.DMA((2,)), pltpu.SemaphoreType.DMA((2,)),
        ],
    )
    return run(state_flat, i0_blk, i1_blk)


def _sc_scatter(m_blk_flat, i0_blk):
    def body(m_hbm, i0_hbm, agg_hbm,
             agg_v, ia, ib2, ma, mb, s_i, s_m):
        wid = lax.axis_index("s") * 2 + lax.axis_index("c")
        base = wid * NCH * CB
        ibufs = (ia, ib2)
        mbufs = (ma, mb)

        def in_start(c, b):
            off = base + c * CB
            pltpu.async_copy(i0_hbm.at[pl.ds(off, CB)], ibufs[b], s_i.at[b])
            pltpu.async_copy(m_hbm.at[pl.ds(off, CB)], mbufs[b], s_m.at[b])

        def in_wait(b):
            pltpu.make_async_copy(i0_hbm.at[pl.ds(0, CB)], ibufs[b], s_i.at[b]).wait()
            pltpu.make_async_copy(m_hbm.at[pl.ds(0, CB)], mbufs[b], s_m.at[b]).wait()

        in_start(0, 0)

        def zero_body(i, _):
            agg_v[pl.ds(i * 16, 16)] = jnp.zeros((16,), jnp.float32)
            return 0

        lax.fori_loop(0, RN // 16, zero_body, 0)

        def compute(b):
            for r in range(RPW):
                rbase = jnp.int32(r * NP)

                def s_body(g, _, r=r, rbase=rbase, b=b):
                    off = r * C + g * 16
                    iv = ibufs[b][pl.ds(off, 16)] + rbase
                    vals = mbufs[b][pl.ds(off, 16)]
                    plsc.addupdate_scatter(agg_v, [iv], vals)
                    return 0

                lax.fori_loop(0, C // 16, s_body, 0)

        def loop_body(c2, _):
            for b in (0, 1):
                c = c2 * 2 + b
                in_wait(b)

                @pl.when(c < NCH - 1)
                def _(c=c, b=b):
                    in_start(c + 1, 1 - b)

                compute(b)
            return 0

        lax.fori_loop(0, NCH // 2, loop_body, 0)
        pltpu.sync_copy(agg_v, agg_hbm.at[pl.ds(wid * RN, RN)])

    f32 = jnp.float32
    i32 = jnp.int32
    run = pl.kernel(
        body,
        out_type=jax.ShapeDtypeStruct((F * NP,), f32),
        mesh=_sc_mesh(),
        compiler_params=_SC_PARAMS,
        scratch_types=[
            pltpu.VMEM((RN,), f32),
            pltpu.VMEM((CB,), i32), pltpu.VMEM((CB,), i32),
            pltpu.VMEM((CB,), f32), pltpu.VMEM((CB,), f32),
            pltpu.SemaphoreType.DMA((2,)), pltpu.SemaphoreType.DMA((2,)),
        ],
    )
    return run(m_blk_flat, i0_blk)


def _msg_body(t0_ref, t1_ref, wa_ref, wb_ref, bm_ref, out_ref):
    x0 = jnp.reshape(t0_ref[...], (F, EBLK))
    x1 = jnp.reshape(t1_ref[...], (F, EBLK))
    acc = jnp.dot(wa_ref[...], x0, preferred_element_type=jnp.float32,
                  precision=jax.lax.Precision.HIGHEST)
    acc += jnp.dot(wb_ref[...], x1, preferred_element_type=jnp.float32,
                   precision=jax.lax.Precision.HIGHEST)
    acc += bm_ref[...]
    out_ref[...] = jnp.reshape(acc, (NW, 1, RPW, EBLK))


def _msg_matmul(tmp0_blk, tmp1_blk, WmA, WmB, bm2):
    return pl.pallas_call(
        _msg_body,
        grid=(NCH,),
        in_specs=[
            pl.BlockSpec((NW, 1, RPW, EBLK), lambda c: (0, c, 0, 0)),
            pl.BlockSpec((NW, 1, RPW, EBLK), lambda c: (0, c, 0, 0)),
            pl.BlockSpec((F, F), lambda c: (0, 0)),
            pl.BlockSpec((F, F), lambda c: (0, 0)),
            pl.BlockSpec((F, 1), lambda c: (0, 0)),
        ],
        out_specs=pl.BlockSpec((NW, 1, RPW, EBLK), lambda c: (0, c, 0, 0)),
        out_shape=jax.ShapeDtypeStruct((NW, NCH, RPW, EBLK), jnp.float32),
    )(tmp0_blk, tmp1_blk, WmA, WmB, bm2)


def _gru_body(agg_ref, st_ref, wih_ref, whh_ref, bih_ref, bhh_ref, out_ref):
    gi = jnp.dot(wih_ref[...], agg_ref[...], preferred_element_type=jnp.float32,
                 precision=jax.lax.Precision.HIGHEST)
    gi += bih_ref[...]
    gh = jnp.dot(whh_ref[...], st_ref[...], preferred_element_type=jnp.float32,
                 precision=jax.lax.Precision.HIGHEST)
    gh += bhh_ref[...]
    r = jax.nn.sigmoid(gi[0:F, :] + gh[0:F, :])
    z = jax.nn.sigmoid(gi[F:2 * F, :] + gh[F:2 * F, :])
    n = jnp.tanh(gi[2 * F:, :] + r * gh[2 * F:, :])
    out_ref[...] = (1.0 - z) * n + z * st_ref[...]


def _gru(agg_t, state_t, W_ih, W_hh, bih2, bhh2):
    return pl.pallas_call(
        _gru_body,
        grid=(NP // NB,),
        in_specs=[
            pl.BlockSpec((F, NB), lambda j: (0, j)),
            pl.BlockSpec((F, NB), lambda j: (0, j)),
            pl.BlockSpec((3 * F, F), lambda j: (0, 0)),
            pl.BlockSpec((3 * F, F), lambda j: (0, 0)),
            pl.BlockSpec((3 * F, 1), lambda j: (0, 0)),
            pl.BlockSpec((3 * F, 1), lambda j: (0, 0)),
        ],
        out_specs=pl.BlockSpec((F, NB), lambda j: (0, j)),
        out_shape=jax.ShapeDtypeStruct((F, NP), jnp.float32),
    )(agg_t, state_t, W_ih, W_hh, bih2, bhh2)


def _head_body(st_ref, w1_ref, b1_ref, w2_ref, b2_ref, w3_ref, b3_ref,
               out_ref, acc_ref):
    j = pl.program_id(0)
    nb = pl.num_programs(0)

    col = jax.lax.broadcasted_iota(jnp.int32, (NB, 1), 0) + j * NB
    ones = jnp.where(col < N, 1.0, 0.0).astype(jnp.float32)
    part = jnp.dot(st_ref[...], ones, preferred_element_type=jnp.float32,
                   precision=jax.lax.Precision.HIGHEST)

    @pl.when(j == 0)
    def _():
        acc_ref[...] = jnp.zeros_like(acc_ref)

    acc_ref[...] += part

    @pl.when(j == nb - 1)
    def _():
        feat = acc_ref[...]
        h1 = jnp.maximum(jnp.dot(w1_ref[...], feat,
                                 preferred_element_type=jnp.float32,
                                 precision=jax.lax.Precision.HIGHEST)
                         + b1_ref[...], 0.0)
        h2 = jnp.maximum(jnp.dot(w2_ref[...], h1,
                                 preferred_element_type=jnp.float32,
                                 precision=jax.lax.Precision.HIGHEST)
                         + b2_ref[...], 0.0)
        out_ref[...] = (jnp.dot(w3_ref[...], h2,
                                preferred_element_type=jnp.float32,
                                precision=jax.lax.Precision.HIGHEST)
                        + b3_ref[...])


def _head(state_t, W1, b12, W2, b22, W3, b32):
    out = pl.pallas_call(
        _head_body,
        grid=(NP // NB,),
        in_specs=[
            pl.BlockSpec((F, NB), lambda j: (0, j)),
            pl.BlockSpec((F, F), lambda j: (0, 0)),
            pl.BlockSpec((F, 1), lambda j: (0, 0)),
            pl.BlockSpec((F // 2, F), lambda j: (0, 0)),
            pl.BlockSpec((F // 2, 1), lambda j: (0, 0)),
            pl.BlockSpec((1, F // 2), lambda j: (0, 0)),
            pl.BlockSpec((1, 1), lambda j: (0, 0)),
        ],
        out_specs=pl.BlockSpec((1, 1), lambda j: (0, 0)),
        out_shape=jax.ShapeDtypeStruct((1, 1), jnp.float32),
        scratch_shapes=[pltpu.VMEM((F, 1), jnp.float32)],
    )(state_t, W1, b12, W2, b22, W3, b32)
    return out.reshape((1,))


def _to_worker_blocked(idx_2d):
    return (idx_2d.reshape(NW, RPW, NCH, C)
            .transpose(0, 2, 1, 3)
            .reshape(-1))


def kernel(link_state, pair, Wm, bm, W_ih, W_hh, b_ih, b_hh, W1, b1, W2, b2, W3, b3):
    i0_blk = _to_worker_blocked(pair[0::2].T.astype(jnp.int32))
    i1_blk = _to_worker_blocked(pair[1::2].T.astype(jnp.int32))
    WmA = Wm[:, :F]
    WmB = Wm[:, F:]
    bm2 = bm.reshape(F, 1)
    bih2 = b_ih.reshape(3 * F, 1)
    bhh2 = b_hh.reshape(3 * F, 1)
    b12 = b1.reshape(F, 1)
    b22 = b2.reshape(F // 2, 1)
    b32 = b3.reshape(1, 1)

    state_t = jnp.pad(link_state.T, ((0, 0), (0, NP - N)))

    for _ in range(T):
        tmp0_f, tmp1_f = _sc_gather(state_t.reshape(-1), i0_blk, i1_blk)
        m_blk = _msg_matmul(tmp0_f.reshape(NW, NCH, RPW, C),
                            tmp1_f.reshape(NW, NCH, RPW, C),
                            WmA, WmB, bm2)
        agg_f = _sc_scatter(m_blk.reshape(-1), i0_blk)
        state_t = _gru(agg_f.reshape(F, NP), state_t, W_ih, W_hh, bih2, bhh2)

    return _head(state_t, W1, b12, W2, b22, W3, b32)


# speedup vs baseline: 582.6809x; 1.0823x over previous
import functools

import jax
import jax.numpy as jnp
from jax import lax
from jax.experimental import pallas as pl
from jax.experimental.pallas import tpu as pltpu
from jax.experimental.pallas import tpu_sc as plsc

T = 4
F = 256
N = 10000
E = 160000

NW = 32
RPW = 8
C = 640
CB = RPW * C
NCH = E // C
NP = 10240
RN = RPW * NP
NB = 1280
EBLK = C


def _sc_mesh():
    return plsc.VectorSubcoreMesh(core_axis_name="c", subcore_axis_name="s",
                                  num_cores=2, num_subcores=16)


_SC_PARAMS = pltpu.CompilerParams(needs_layout_passes=False)


def _sc_gather(state_flat, i0_blk, i1_blk):
    def body(st_hbm, i0_hbm, i1_hbm, t0_hbm, t1_hbm,
             st_v, i0a, i0b, i1a, i1b, o0a, o0b, o1a, o1b,
             s_st, s_i0, s_i1, s_o0, s_o1):
        wid = lax.axis_index("s") * 2 + lax.axis_index("c")
        base = wid * NCH * CB
        ib0 = (i0a, i0b)
        ib1 = (i1a, i1b)
        ob0 = (o0a, o0b)
        ob1 = (o1a, o1b)

        def idx_start(c, b):
            off = base + c * CB
            pltpu.async_copy(i0_hbm.at[pl.ds(off, CB)], ib0[b], s_i0.at[b])
            pltpu.async_copy(i1_hbm.at[pl.ds(off, CB)], ib1[b], s_i1.at[b])

        def idx_wait(b):
            pltpu.make_async_copy(i0_hbm.at[pl.ds(0, CB)], ib0[b], s_i0.at[b]).wait()
            pltpu.make_async_copy(i1_hbm.at[pl.ds(0, CB)], ib1[b], s_i1.at[b]).wait()

        def out_start(c, b):
            off = base + c * CB
            pltpu.async_copy(ob0[b], t0_hbm.at[pl.ds(off, CB)], s_o0.at[b])
            pltpu.async_copy(ob1[b], t1_hbm.at[pl.ds(off, CB)], s_o1.at[b])

        def out_wait(b):
            pltpu.make_async_copy(ob0[b], t0_hbm.at[pl.ds(0, CB)], s_o0.at[b]).wait()
            pltpu.make_async_copy(ob1[b], t1_hbm.at[pl.ds(0, CB)], s_o1.at[b]).wait()

        pltpu.async_copy(st_hbm.at[pl.ds(wid * RN, RN)], st_v, s_st)
        idx_start(0, 0)
        pltpu.make_async_copy(st_hbm.at[pl.ds(0, RN)], st_v, s_st).wait()

        def compute(b):
            def g_body(g, _, b=b):
                off = g * 64
                for u in range(4):
                    o = off + u * 16
                    iv0 = ib0[b][pl.ds(o, 16)]
                    ob0[b][pl.ds(o, 16)] = plsc.load_gather(st_v, [iv0])
                    iv1 = ib1[b][pl.ds(o, 16)]
                    ob1[b][pl.ds(o, 16)] = plsc.load_gather(st_v, [iv1])
                return 0

            lax.fori_loop(0, CB // 64, g_body, 0)

        def loop_body(c2, _):
            for b in (0, 1):
                c = c2 * 2 + b
                idx_wait(b)

                @pl.when(c < NCH - 1)
                def _(c=c, b=b):
                    idx_start(c + 1, 1 - b)

                @pl.when(c2 >= 1)
                def _(b=b):
                    out_wait(b)

                compute(b)
                out_start(c, b)
            return 0

        lax.fori_loop(0, NCH // 2, loop_body, 0)
        out_wait(0)
        out_wait(1)

    f32 = jnp.float32
    i32 = jnp.int32
    run = pl.kernel(
        body,
        out_type=[jax.ShapeDtypeStruct((F * E,), f32),
                  jax.ShapeDtypeStruct((F * E,), f32)],
        mesh=_sc_mesh(),
        compiler_params=_SC_PARAMS,
        scratch_types=[
            pltpu.VMEM((RN,), f32),
            pltpu.VMEM((CB,), i32), pltpu.VMEM((CB,), i32),
            pltpu.VMEM((CB,), i32), pltpu.VMEM((CB,), i32),
            pltpu.VMEM((CB,), f32), pltpu.VMEM((CB,), f32),
            pltpu.VMEM((CB,), f32), pltpu.VMEM((CB,), f32),
            pltpu.SemaphoreType.DMA,
            pltpu.SemaphoreType.DMA((2,)), pltpu.SemaphoreType.DMA((2,)),
            pltpu.SemaphoreType.DMA((2,)), pltpu.SemaphoreType.DMA((2,)),
        ],
    )
    return run(state_flat, i0_blk, i1_blk)


def _sc_scatter(m_blk_flat, i0_blk):
    def body(m_hbm, i0_hbm, agg_hbm,
             agg_v, ia, ib2, ma, mb, s_i, s_m):
        wid = lax.axis_index("s") * 2 + lax.axis_index("c")
        base = wid * NCH * CB
        ibufs = (ia, ib2)
        mbufs = (ma, mb)

        def in_start(c, b):
            off = base + c * CB
            pltpu.async_copy(i0_hbm.at[pl.ds(off, CB)], ibufs[b], s_i.at[b])
            pltpu.async_copy(m_hbm.at[pl.ds(off, CB)], mbufs[b], s_m.at[b])

        def in_wait(b):
            pltpu.make_async_copy(i0_hbm.at[pl.ds(0, CB)], ibufs[b], s_i.at[b]).wait()
            pltpu.make_async_copy(m_hbm.at[pl.ds(0, CB)], mbufs[b], s_m.at[b]).wait()

        in_start(0, 0)

        def zero_body(i, _):
            agg_v[pl.ds(i * 16, 16)] = jnp.zeros((16,), jnp.float32)
            return 0

        lax.fori_loop(0, RN // 16, zero_body, 0)

        def compute(b):
            def s_body(g, _, b=b):
                off = g * 64
                for u in range(4):
                    o = off + u * 16
                    iv = ibufs[b][pl.ds(o, 16)]
                    vals = mbufs[b][pl.ds(o, 16)]
                    plsc.addupdate_scatter(agg_v, [iv], vals)
                return 0

            lax.fori_loop(0, CB // 64, s_body, 0)

        def loop_body(c2, _):
            for b in (0, 1):
                c = c2 * 2 + b
                in_wait(b)

                @pl.when(c < NCH - 1)
                def _(c=c, b=b):
                    in_start(c + 1, 1 - b)

                compute(b)
            return 0

        lax.fori_loop(0, NCH // 2, loop_body, 0)
        pltpu.sync_copy(agg_v, agg_hbm.at[pl.ds(wid * RN, RN)])

    f32 = jnp.float32
    i32 = jnp.int32
    run = pl.kernel(
        body,
        out_type=jax.ShapeDtypeStruct((F * NP,), f32),
        mesh=_sc_mesh(),
        compiler_params=_SC_PARAMS,
        scratch_types=[
            pltpu.VMEM((RN,), f32),
            pltpu.VMEM((CB,), i32), pltpu.VMEM((CB,), i32),
            pltpu.VMEM((CB,), f32), pltpu.VMEM((CB,), f32),
            pltpu.SemaphoreType.DMA((2,)), pltpu.SemaphoreType.DMA((2,)),
        ],
    )
    return run(m_blk_flat, i0_blk)


def _msg_body(t0_ref, t1_ref, wa_ref, wb_ref, bm_ref, out_ref):
    x0 = jnp.reshape(t0_ref[...], (F, EBLK)).astype(jnp.bfloat16)
    x1 = jnp.reshape(t1_ref[...], (F, EBLK)).astype(jnp.bfloat16)
    acc = jnp.dot(wa_ref[...], x0, preferred_element_type=jnp.float32)
    acc += jnp.dot(wb_ref[...], x1, preferred_element_type=jnp.float32)
    acc += bm_ref[...]
    out_ref[...] = jnp.reshape(acc, (NW, 1, RPW, EBLK))


def _msg_matmul(tmp0_blk, tmp1_blk, WmA, WmB, bm2):
    wspec = pl.BlockSpec((F, F), lambda c: (0, 0))
    return pl.pallas_call(
        _msg_body,
        grid=(NCH,),
        in_specs=[
            pl.BlockSpec((NW, 1, RPW, EBLK), lambda c: (0, c, 0, 0)),
            pl.BlockSpec((NW, 1, RPW, EBLK), lambda c: (0, c, 0, 0)),
            wspec, wspec,
            pl.BlockSpec((F, 1), lambda c: (0, 0)),
        ],
        out_specs=pl.BlockSpec((NW, 1, RPW, EBLK), lambda c: (0, c, 0, 0)),
        out_shape=jax.ShapeDtypeStruct((NW, NCH, RPW, EBLK), jnp.float32),
    )(tmp0_blk, tmp1_blk, WmA, WmB, bm2)


def _gru_body(agg_ref, st_ref, wih_ref, whh_ref, bih_ref, bhh_ref, out_ref):
    gi = jnp.dot(wih_ref[...], agg_ref[...].astype(jnp.bfloat16),
                 preferred_element_type=jnp.float32)
    gi += bih_ref[...]
    gh = jnp.dot(whh_ref[...], st_ref[...].astype(jnp.bfloat16),
                 preferred_element_type=jnp.float32)
    gh += bhh_ref[...]
    r = jax.nn.sigmoid(gi[0:F, :] + gh[0:F, :])
    z = jax.nn.sigmoid(gi[F:2 * F, :] + gh[F:2 * F, :])
    n = jnp.tanh(gi[2 * F:, :] + r * gh[2 * F:, :])
    out_ref[...] = (1.0 - z) * n + z * st_ref[...]


def _gru(agg_t, state_t, W_ih, W_hh, bih2, bhh2):
    return pl.pallas_call(
        _gru_body,
        grid=(NP // NB,),
        in_specs=[
            pl.BlockSpec((F, NB), lambda j: (0, j)),
            pl.BlockSpec((F, NB), lambda j: (0, j)),
            pl.BlockSpec((3 * F, F), lambda j: (0, 0)),
            pl.BlockSpec((3 * F, F), lambda j: (0, 0)),
            pl.BlockSpec((3 * F, 1), lambda j: (0, 0)),
            pl.BlockSpec((3 * F, 1), lambda j: (0, 0)),
        ],
        out_specs=pl.BlockSpec((F, NB), lambda j: (0, j)),
        out_shape=jax.ShapeDtypeStruct((F, NP), jnp.float32),
    )(agg_t, state_t, W_ih, W_hh, bih2, bhh2)


def _head_body(st_ref, w1_ref, b1_ref, w2_ref, b2_ref, w3_ref, b3_ref,
               out_ref, acc_ref):
    j = pl.program_id(0)
    nb = pl.num_programs(0)

    col = jax.lax.broadcasted_iota(jnp.int32, (NB, 1), 0) + j * NB
    ones = jnp.where(col < N, 1.0, 0.0).astype(jnp.float32)
    part = jnp.dot(st_ref[...], ones, preferred_element_type=jnp.float32,
                   precision=jax.lax.Precision.HIGHEST)

    @pl.when(j == 0)
    def _():
        acc_ref[...] = jnp.zeros_like(acc_ref)

    acc_ref[...] += part

    @pl.when(j == nb - 1)
    def _():
        feat = acc_ref[...]
        h1 = jnp.maximum(jnp.dot(w1_ref[...].astype(jnp.bfloat16),
                                 feat.astype(jnp.bfloat16),
                                 preferred_element_type=jnp.float32)
                         + b1_ref[...], 0.0)
        h2 = jnp.maximum(jnp.dot(w2_ref[...].astype(jnp.bfloat16),
                                 h1.astype(jnp.bfloat16),
                                 preferred_element_type=jnp.float32)
                         + b2_ref[...], 0.0)
        out_ref[...] = (jnp.dot(w3_ref[...].astype(jnp.bfloat16),
                                h2.astype(jnp.bfloat16),
                                preferred_element_type=jnp.float32)
                        + b3_ref[...])


def _head(state_t, W1, b12, W2, b22, W3, b32):
    out = pl.pallas_call(
        _head_body,
        grid=(NP // NB,),
        in_specs=[
            pl.BlockSpec((F, NB), lambda j: (0, j)),
            pl.BlockSpec((F, F), lambda j: (0, 0)),
            pl.BlockSpec((F, 1), lambda j: (0, 0)),
            pl.BlockSpec((F // 2, F), lambda j: (0, 0)),
            pl.BlockSpec((F // 2, 1), lambda j: (0, 0)),
            pl.BlockSpec((1, F // 2), lambda j: (0, 0)),
            pl.BlockSpec((1, 1), lambda j: (0, 0)),
        ],
        out_specs=pl.BlockSpec((1, 1), lambda j: (0, 0)),
        out_shape=jax.ShapeDtypeStruct((1, 1), jnp.float32),
        scratch_shapes=[pltpu.VMEM((F, 1), jnp.float32)],
    )(state_t, W1, b12, W2, b22, W3, b32)
    return out.reshape((1,))


def _to_worker_blocked(idx_2d):
    blk = idx_2d.reshape(NW, RPW, NCH, C).transpose(0, 2, 1, 3)
    off = (jnp.arange(RPW, dtype=jnp.int32) * NP)[None, None, :, None]
    return (blk + off).reshape(-1)


def kernel(link_state, pair, Wm, bm, W_ih, W_hh, b_ih, b_hh, W1, b1, W2, b2, W3, b3):
    i0_blk = _to_worker_blocked(pair[0::2].T.astype(jnp.int32))
    i1_blk = _to_worker_blocked(pair[1::2].T.astype(jnp.int32))
    WmA = Wm[:, :F].astype(jnp.bfloat16)
    WmB = Wm[:, F:].astype(jnp.bfloat16)
    bm2 = bm.reshape(F, 1)
    Wih16 = W_ih.astype(jnp.bfloat16)
    Whh16 = W_hh.astype(jnp.bfloat16)
    bih2 = b_ih.reshape(3 * F, 1)
    bhh2 = b_hh.reshape(3 * F, 1)
    b12 = b1.reshape(F, 1)
    b22 = b2.reshape(F // 2, 1)
    b32 = b3.reshape(1, 1)

    state_t = jnp.pad(link_state.T, ((0, 0), (0, NP - N)))

    for _ in range(T):
        tmp0_f, tmp1_f = _sc_gather(state_t.reshape(-1), i0_blk, i1_blk)
        m_blk = _msg_matmul(tmp0_f.reshape(NW, NCH, RPW, C),
                            tmp1_f.reshape(NW, NCH, RPW, C),
                            WmA, WmB, bm2)
        agg_f = _sc_scatter(m_blk.reshape(-1), i0_blk)
        state_t = _gru(agg_f.reshape(F, NP), state_t, Wih16, Whh16, bih2, bhh2)

    return _head(state_t, W1, b12, W2, b22, W3, b32)
